# baseline (speedup 1.0000x reference)
import functools

import jax
import jax.numpy as jnp
from jax import lax
from jax.experimental import pallas as pl
from jax.experimental.pallas import tpu as pltpu
from jax.experimental.pallas import tpu_sc as plsc

B, T = 1024, 50
E, H = 128, 256
G = 3 * H

_NT = 2
_TCH = T // _NT

_NC, _NS = 2, 16
_NW = _NC * _NS
_ROWS = B * _TCH
_RPW = _ROWS // _NW
_CHUNK = 80
_NCHUNK = _RPW // _CHUNK


def _sc_gather(emb, idx):
    mesh = plsc.VectorSubcoreMesh(core_axis_name="c", subcore_axis_name="s")

    @functools.partial(
        pl.kernel,
        out_type=jax.ShapeDtypeStruct((_ROWS, E), jnp.float32),
        mesh=mesh,
        scratch_types=[
            pltpu.VMEM((_NCHUNK, _CHUNK), jnp.int32),
            pltpu.VMEM((_CHUNK, E), jnp.float32),
            pltpu.VMEM((_CHUNK, E), jnp.float32),
            pltpu.SemaphoreType.DMA,
            pltpu.SemaphoreType.DMA,
        ],
    )
    def gather_kernel(emb_hbm, idx_hbm, out_hbm, idx_v, rows0, rows1, sem0, sem1):
        wid = lax.axis_index("s") * _NC + lax.axis_index("c")
        pltpu.sync_copy(idx_hbm.at[wid], idx_v)
        base = wid * _RPW
        bufs = [(rows0, sem0), (rows1, sem1)]
        cps = [None, None]
        cps[0] = pltpu.async_copy(emb_hbm.at[idx_v.at[0]], rows0, sem0)
        for c in range(_NCHUNK):
            buf, _ = bufs[c % 2]
            cps[c % 2].wait()
            if c + 1 < _NCHUNK:
                nbuf, nsem = bufs[(c + 1) % 2]
                cps[(c + 1) % 2] = pltpu.async_copy(
                    emb_hbm.at[idx_v.at[c + 1]], nbuf, nsem)
            pltpu.sync_copy(buf, out_hbm.at[pl.ds(base + c * _CHUNK, _CHUNK)])

    return gather_kernel(emb, idx)


_U = 5
_NG = _TCH // _U


def _gru_body(e_ref, hin_ref, w0i_ref, w0h_ref, w1i_ref, w1h_ref,
              bi0_ref, bh0_ref, bi1_ref, bh1_ref,
              out_ref, hid_ref, h0_ref, h1_ref):
    g = pl.program_id(0)

    @pl.when(g == 0)
    def _():
        h0_ref[...] = hin_ref[0]
        h1_ref[...] = hin_ref[1]

    def cell(xt, h, wi_ref, wh_ref, bi_ref, bh_ref):
        gi = jnp.dot(xt, wi_ref[...], preferred_element_type=jnp.float32) + bi_ref[...]
        gh = jnp.dot(h, wh_ref[...], preferred_element_type=jnp.float32) + bh_ref[...]
        r = jax.nn.sigmoid(gi[:, :H] + gh[:, :H])
        z = jax.nn.sigmoid(gi[:, H:2 * H] + gh[:, H:2 * H])
        n = jnp.tanh(gi[:, 2 * H:] + r * gh[:, 2 * H:])
        return (1.0 - z) * n + z * h

    h0 = h0_ref[...]
    h1 = h1_ref[...]
    for u in range(_U):
        h0 = cell(e_ref[u], h0, w0i_ref, w0h_ref, bi0_ref, bh0_ref)
        h1 = cell(h0, h1, w1i_ref, w1h_ref, bi1_ref, bh1_ref)
        out_ref[u] = h1
        if u == _U - 1:
            @pl.when(g == _NG - 1)
            def _():
                hid_ref[0] = h0
                hid_ref[1] = h1
    h0_ref[...] = h0
    h1_ref[...] = h1


def _gru_call(e, hin, w0i, w0h, w1i, w1h, bi0, bh0, bi1, bh1):
    full = lambda shape: pl.BlockSpec(shape, lambda t: (0,) * len(shape))
    return pl.pallas_call(
        _gru_body,
        grid=(_NG,),
        in_specs=[
            pl.BlockSpec((_U, B, E), lambda t: (t, 0, 0)),
            full((2, B, H)),
            full((E, G)), full((H, G)), full((H, G)), full((H, G)),
            full((1, G)), full((1, G)), full((1, G)), full((1, G)),
        ],
        out_specs=[
            pl.BlockSpec((_U, B, H), lambda t: (t, 0, 0)),
            pl.BlockSpec((2, B, H), lambda t: (0, 0, 0)),
        ],
        out_shape=[
            jax.ShapeDtypeStruct((_TCH, B, H), jnp.float32),
            jax.ShapeDtypeStruct((2, B, H), jnp.float32),
        ],
        scratch_shapes=[
            pltpu.VMEM((B, H), jnp.float32),
            pltpu.VMEM((B, H), jnp.float32),
        ],
        compiler_params=pltpu.CompilerParams(
            dimension_semantics=("arbitrary",),
        ),
    )(e, hin, w0i, w0h, w1i, w1h, bi0, bh0, bi1, bh1)


def kernel(x, emb, W_ih_l0, W_hh_l0, b_ih_l0, b_hh_l0,
           W_ih_l1, W_hh_l1, b_ih_l1, b_hh_l1):
    idx = x.T.reshape(_NT, _NW, _NCHUNK, _CHUNK).astype(jnp.int32)
    ws = (W_ih_l0.T, W_hh_l0.T, W_ih_l1.T, W_hh_l1.T,
          b_ih_l0.reshape(1, G), b_hh_l0.reshape(1, G),
          b_ih_l1.reshape(1, G), b_hh_l1.reshape(1, G))
    es = [_sc_gather(emb, idx[ct]).reshape(_TCH, B, E) for ct in range(_NT)]
    hid = jnp.zeros((2, B, H), jnp.float32)
    outs = []
    for ct in range(_NT):
        out_c, hid = _gru_call(es[ct], hid, *ws)
        outs.append(out_c)
    out = jnp.concatenate(outs, axis=0)
    return out.transpose(1, 0, 2), hid


# speedup vs baseline: 1.1122x; 1.1122x over previous
import functools

import jax
import jax.numpy as jnp
from jax import lax
from jax.experimental import pallas as pl
from jax.experimental.pallas import tpu as pltpu
from jax.experimental.pallas import tpu_sc as plsc

B, T = 1024, 50
E, H = 128, 256
G = 3 * H

_NC, _NS = 2, 16
_NW = _NC * _NS
_ROWS = B * T
_RPW = _ROWS // _NW
_CHUNK = 80
_NCHUNK = _RPW // _CHUNK


def _sc_gather(emb, idx):
    mesh = plsc.VectorSubcoreMesh(core_axis_name="c", subcore_axis_name="s")

    @functools.partial(
        pl.kernel,
        out_type=jax.ShapeDtypeStruct((_ROWS, E), jnp.float32),
        mesh=mesh,
        scratch_types=[
            pltpu.VMEM((_NCHUNK, _CHUNK), jnp.int32),
            pltpu.VMEM((_CHUNK, E), jnp.float32),
            pltpu.VMEM((_CHUNK, E), jnp.float32),
            pltpu.SemaphoreType.DMA,
            pltpu.SemaphoreType.DMA,
        ],
    )
    def gather_kernel(emb_hbm, idx_hbm, out_hbm, idx_v, rows0, rows1, sem0, sem1):
        wid = lax.axis_index("s") * _NC + lax.axis_index("c")
        pltpu.sync_copy(idx_hbm.at[wid], idx_v)
        base = wid * _RPW
        bufs = [(rows0, sem0), (rows1, sem1)]
        cps = [None, None]
        cps[0] = pltpu.async_copy(emb_hbm.at[idx_v.at[0]], rows0, sem0)
        for c in range(_NCHUNK):
            buf, _ = bufs[c % 2]
            cps[c % 2].wait()
            if c + 1 < _NCHUNK:
                nbuf, nsem = bufs[(c + 1) % 2]
                cps[(c + 1) % 2] = pltpu.async_copy(
                    emb_hbm.at[idx_v.at[c + 1]], nbuf, nsem)
            pltpu.sync_copy(buf, out_hbm.at[pl.ds(base + c * _CHUNK, _CHUNK)])

    return gather_kernel(emb, idx)


_U = 5
_NG = T // _U


def _gru_body(e_ref, w0i_ref, w0h_ref, w1i_ref, w1h_ref,
              bi0_ref, bh0_ref, bi1_ref, bh1_ref,
              out_ref, hid_ref, h0_ref, h1_ref):
    g = pl.program_id(0)

    @pl.when(g == 0)
    def _():
        h0_ref[...] = jnp.zeros_like(h0_ref)
        h1_ref[...] = jnp.zeros_like(h1_ref)

    def cell(xt, h, wi_ref, wh_ref, bi_ref, bh_ref):
        gi = jnp.dot(xt, wi_ref[...], preferred_element_type=jnp.float32) + bi_ref[...]
        gh = jnp.dot(h, wh_ref[...], preferred_element_type=jnp.float32) + bh_ref[...]
        tr = jnp.tanh((gi[:, :H] + gh[:, :H]) * 0.5)
        tz = jnp.tanh((gi[:, H:2 * H] + gh[:, H:2 * H]) * 0.5)
        n = jnp.tanh(gi[:, 2 * H:] + (0.5 * (tr + 1.0)) * gh[:, 2 * H:])
        return 0.5 * ((n + h) + tz * (h - n))

    h0 = h0_ref[...]
    h1 = h1_ref[...]
    for u in range(_U):
        h0 = cell(e_ref[u], h0, w0i_ref, w0h_ref, bi0_ref, bh0_ref)
        h1 = cell(h0, h1, w1i_ref, w1h_ref, bi1_ref, bh1_ref)
        out_ref[u] = h1
        if u == _U - 1:
            @pl.when(g == _NG - 1)
            def _():
                hid_ref[0] = h0
                hid_ref[1] = h1
    h0_ref[...] = h0
    h1_ref[...] = h1


def _gru_call(e, w0i, w0h, w1i, w1h, bi0, bh0, bi1, bh1):
    full = lambda shape: pl.BlockSpec(shape, lambda t: (0,) * len(shape))
    return pl.pallas_call(
        _gru_body,
        grid=(_NG,),
        in_specs=[
            pl.BlockSpec((_U, B, E), lambda t: (t, 0, 0)),
            full((E, G)), full((H, G)), full((H, G)), full((H, G)),
            full((1, G)), full((1, G)), full((1, G)), full((1, G)),
        ],
        out_specs=[
            pl.BlockSpec((_U, B, H), lambda t: (t, 0, 0)),
            pl.BlockSpec((2, B, H), lambda t: (0, 0, 0)),
        ],
        out_shape=[
            jax.ShapeDtypeStruct((T, B, H), jnp.float32),
            jax.ShapeDtypeStruct((2, B, H), jnp.float32),
        ],
        scratch_shapes=[
            pltpu.VMEM((B, H), jnp.float32),
            pltpu.VMEM((B, H), jnp.float32),
        ],
        compiler_params=pltpu.CompilerParams(
            dimension_semantics=("arbitrary",),
        ),
    )(e, w0i, w0h, w1i, w1h, bi0, bh0, bi1, bh1)


def kernel(x, emb, W_ih_l0, W_hh_l0, b_ih_l0, b_hh_l0,
           W_ih_l1, W_hh_l1, b_ih_l1, b_hh_l1):
    idx = x.T.reshape(_NW, _NCHUNK, _CHUNK).astype(jnp.int32)
    e = _sc_gather(emb, idx).reshape(T, B, E)
    out, hid = _gru_call(
        e,
        W_ih_l0.T, W_hh_l0.T, W_ih_l1.T, W_hh_l1.T,
        b_ih_l0.reshape(1, G), b_hh_l0.reshape(1, G),
        b_ih_l1.reshape(1, G), b_hh_l1.reshape(1, G),
    )
    return out.transpose(1, 0, 2), hid


# speedup vs baseline: 1.1737x; 1.0553x over previous
import functools

import jax
import jax.numpy as jnp
from jax import lax
from jax.experimental import pallas as pl
from jax.experimental.pallas import tpu as pltpu
from jax.experimental.pallas import tpu_sc as plsc

B, T = 1024, 50
E, H = 128, 256
G = 3 * H

_NC, _NS = 2, 16
_NW = _NC * _NS
_ROWS = B * T
_RPW = _ROWS // _NW
_CHUNK = 80
_NCHUNK = _RPW // _CHUNK


def _sc_gather(emb, idx):
    mesh = plsc.VectorSubcoreMesh(core_axis_name="c", subcore_axis_name="s")

    @functools.partial(
        pl.kernel,
        out_type=jax.ShapeDtypeStruct((_ROWS, E), jnp.float32),
        mesh=mesh,
        scratch_types=[
            pltpu.VMEM((_NCHUNK, _CHUNK), jnp.int32),
            pltpu.VMEM((_CHUNK, E), jnp.float32),
            pltpu.VMEM((_CHUNK, E), jnp.float32),
            pltpu.SemaphoreType.DMA,
            pltpu.SemaphoreType.DMA,
        ],
    )
    def gather_kernel(emb_hbm, idx_hbm, out_hbm, idx_v, rows0, rows1, sem0, sem1):
        wid = lax.axis_index("s") * _NC + lax.axis_index("c")
        pltpu.sync_copy(idx_hbm.at[wid], idx_v)
        base = wid * _RPW
        bufs = [(rows0, sem0), (rows1, sem1)]
        cps = [None, None]
        cps[0] = pltpu.async_copy(emb_hbm.at[idx_v.at[0]], rows0, sem0)
        for c in range(_NCHUNK):
            buf, _ = bufs[c % 2]
            cps[c % 2].wait()
            if c + 1 < _NCHUNK:
                nbuf, nsem = bufs[(c + 1) % 2]
                cps[(c + 1) % 2] = pltpu.async_copy(
                    emb_hbm.at[idx_v.at[c + 1]], nbuf, nsem)
            pltpu.sync_copy(buf, out_hbm.at[pl.ds(base + c * _CHUNK, _CHUNK)])

    return gather_kernel(emb, idx)


_U = 10
_NG = T // _U


def _gru_body(e_ref, w0i_ref, w0h_ref, w1i_ref, w1h_ref,
              bi0_ref, bh0_ref, bi1_ref, bh1_ref,
              out_ref, hid_ref, h0_ref, h1_ref):
    g = pl.program_id(0)

    @pl.when(g == 0)
    def _():
        h0_ref[...] = jnp.zeros_like(h0_ref)
        h1_ref[...] = jnp.zeros_like(h1_ref)

    def cell(xt, h, wi_ref, wh_ref, bi_ref, bh_ref):
        gi = jnp.dot(xt, wi_ref[...], preferred_element_type=jnp.float32) + bi_ref[...]
        gh = jnp.dot(h, wh_ref[...], preferred_element_type=jnp.float32) + bh_ref[...]
        r = jax.nn.sigmoid(gi[:, :H] + gh[:, :H])
        z = jax.nn.sigmoid(gi[:, H:2 * H] + gh[:, H:2 * H])
        n = jnp.tanh(gi[:, 2 * H:] + r * gh[:, 2 * H:])
        return (1.0 - z) * n + z * h

    h0 = h0_ref[...]
    h1 = h1_ref[...]
    for u in range(_U):
        h0 = cell(e_ref[u], h0, w0i_ref, w0h_ref, bi0_ref, bh0_ref)
        h1 = cell(h0, h1, w1i_ref, w1h_ref, bi1_ref, bh1_ref)
        out_ref[u] = h1
        if u == _U - 1:
            @pl.when(g == _NG - 1)
            def _():
                hid_ref[0] = h0
                hid_ref[1] = h1
    h0_ref[...] = h0
    h1_ref[...] = h1


def _gru_call(e, w0i, w0h, w1i, w1h, bi0, bh0, bi1, bh1):
    full = lambda shape: pl.BlockSpec(shape, lambda t: (0,) * len(shape))
    return pl.pallas_call(
        _gru_body,
        grid=(_NG,),
        in_specs=[
            pl.BlockSpec((_U, B, E), lambda t: (t, 0, 0)),
            full((E, G)), full((H, G)), full((H, G)), full((H, G)),
            full((1, G)), full((1, G)), full((1, G)), full((1, G)),
        ],
        out_specs=[
            pl.BlockSpec((_U, B, H), lambda t: (t, 0, 0)),
            pl.BlockSpec((2, B, H), lambda t: (0, 0, 0)),
        ],
        out_shape=[
            jax.ShapeDtypeStruct((T, B, H), jnp.float32),
            jax.ShapeDtypeStruct((2, B, H), jnp.float32),
        ],
        scratch_shapes=[
            pltpu.VMEM((B, H), jnp.float32),
            pltpu.VMEM((B, H), jnp.float32),
        ],
        compiler_params=pltpu.CompilerParams(
            dimension_semantics=("arbitrary",),
        ),
    )(e, w0i, w0h, w1i, w1h, bi0, bh0, bi1, bh1)


def kernel(x, emb, W_ih_l0, W_hh_l0, b_ih_l0, b_hh_l0,
           W_ih_l1, W_hh_l1, b_ih_l1, b_hh_l1):
    idx = x.T.reshape(_NW, _NCHUNK, _CHUNK).astype(jnp.int32)
    e = _sc_gather(emb, idx).reshape(T, B, E)
    out, hid = _gru_call(
        e,
        W_ih_l0.T, W_hh_l0.T, W_ih_l1.T, W_hh_l1.T,
        b_ih_l0.reshape(1, G), b_hh_l0.reshape(1, G),
        b_ih_l1.reshape(1, G), b_hh_l1.reshape(1, G),
    )
    return out.transpose(1, 0, 2), hid
